# baseline (speedup 1.0000x reference)
import functools

import jax
import jax.numpy as jnp
from jax import lax
from jax.experimental import pallas as pl
from jax.experimental.pallas import tpu as pltpu
from jax.experimental.pallas import tpu_sc as plsc

NUM_CORES = 2
NUM_SUBCORES = 16
NW = NUM_CORES * NUM_SUBCORES

CHUNK = 128
NBUF = 10
LEAD = NBUF // 2


@functools.partial(jax.jit, static_argnums=(2, 3))
def _emb_lookup(idx, table, nchunk, dim):
    b_total = NW * nchunk * CHUNK
    b_per_w = nchunk * CHUNK
    rounds = nchunk // NBUF

    mesh = plsc.VectorSubcoreMesh(core_axis_name="c", subcore_axis_name="s")

    @functools.partial(
        pl.kernel,
        mesh=mesh,
        out_type=jax.ShapeDtypeStruct((b_total, dim), jnp.float32),
        scratch_types=[
            pltpu.VMEM((nchunk, CHUNK), jnp.int32),
            pltpu.VMEM((NBUF, CHUNK, dim), jnp.float32),
        ] + [pltpu.SemaphoreType.DMA] * (2 * NBUF),
        compiler_params=pltpu.CompilerParams(use_tc_tiling_on_sc=False),
    )
    def emb_kernel(idx_hbm, table_hbm, out_hbm, idx_v, rows_v, *sems):
        gsems = sems[:NBUF]
        wsems = sems[NBUF:]
        wid = lax.axis_index("s") * NUM_CORES + lax.axis_index("c")
        base = wid * b_per_w
        nchunks = nchunk
        pltpu.sync_copy(idx_hbm.at[wid], idx_v)

        for b in range(LEAD):
            pltpu.async_copy(table_hbm.at[idx_v.at[b]], rows_v.at[b], gsems[b])

        def out_dst(j):
            return out_hbm.at[pl.ds(base + j * CHUNK, CHUNK)]

        def body(i, carry):
            for b in range(NBUF):
                j = i * NBUF + b
                pltpu.make_async_copy(
                    table_hbm.at[idx_v.at[j]], rows_v.at[b], gsems[b]
                ).wait()
                pltpu.make_async_copy(
                    rows_v.at[b], out_dst(j), wsems[b]
                ).start()

                jj = j + LEAD
                b2 = (b + LEAD) % NBUF

                @pl.when(jj < nchunks)
                def _():
                    @pl.when(jj >= NBUF)
                    def _():
                        pltpu.make_async_copy(
                            rows_v.at[b2], out_dst(jj - NBUF), wsems[b2]
                        ).wait()

                    pltpu.async_copy(
                        table_hbm.at[idx_v.at[jj]], rows_v.at[b2], gsems[b2]
                    )

            return carry

        lax.fori_loop(0, rounds, body, 0)

        for b in range(NBUF):
            j = b_per_w // CHUNK - NBUF + b
            pltpu.make_async_copy(
                rows_v.at[j % NBUF], out_dst(j), wsems[j % NBUF]
            ).wait()

    return emb_kernel(idx, table)


def kernel(x, weight):
    b, t = x.shape
    dim = weight.shape[1]
    b_total = b * t
    assert b_total % (NW * CHUNK) == 0
    nchunk = b_total // (NW * CHUNK)
    assert nchunk % NBUF == 0
    idx = x.reshape(NW, nchunk, CHUNK).astype(jnp.int32)
    rows = _emb_lookup(idx, weight, nchunk, dim)
    return rows.reshape(b, t, dim)
